# baseline (speedup 1.0000x reference)
import functools

import jax
import jax.numpy as jnp
from jax import lax
from jax.experimental import pallas as pl
from jax.experimental.pallas import tpu as pltpu
from jax.experimental.pallas import tpu_sc as plsc

M = 1_000_000
N = 1_000_000
R = 16
B_IDS = 16384

_info = plsc.get_sparse_core_info()
NC, NS, L = _info.num_cores, _info.num_subcores, _info.num_lanes
NW = NC * NS
BPW = B_IDS // NW


def _make_kernel():
    mesh = plsc.VectorSubcoreMesh(core_axis_name="c", subcore_axis_name="s")

    @functools.partial(
        pl.kernel,
        mesh=mesh,
        out_type=jax.ShapeDtypeStruct((B_IDS,), jnp.float32),
        scratch_types=[
            pltpu.VMEM((BPW,), jnp.int32),
            pltpu.VMEM((BPW,), jnp.int32),
            pltpu.VMEM((BPW, R), jnp.float32),
            pltpu.VMEM((BPW, R), jnp.float32),
            pltpu.VMEM((BPW,), jnp.float32),
            pltpu.SemaphoreType.DMA,
            pltpu.SemaphoreType.DMA,
        ],
        compiler_params=pltpu.CompilerParams(
            needs_layout_passes=False, use_tc_tiling_on_sc=False
        ),
    )
    def k(ids0_hbm, ids1_hbm, a_hbm, b_hbm, out_hbm,
          idx_a, idx_b, rows_a, rows_b, out_v, sem_a, sem_b):
        wid = lax.axis_index("s") * NC + lax.axis_index("c")
        base = wid * BPW
        pltpu.sync_copy(ids0_hbm.at[pl.ds(base, BPW)], idx_a)
        pltpu.sync_copy(ids1_hbm.at[pl.ds(base, BPW)], idx_b)
        ca = pltpu.async_copy(a_hbm.at[idx_a], rows_a, sem_a)
        cb = pltpu.async_copy(b_hbm.at[idx_b], rows_b, sem_b)
        ca.wait()
        cb.wait()

        def body(g, carry):
            i0 = g * L
            row_ix = i0 + lax.iota(jnp.int32, L)
            acc = jnp.zeros((L,), jnp.float32)
            for r in range(R):
                col_ix = jnp.full((L,), r, jnp.int32)
                va = plsc.load_gather(rows_a, [row_ix, col_ix])
                vb = plsc.load_gather(rows_b, [row_ix, col_ix])
                acc = acc + va * vb
            out_v[pl.ds(i0, L)] = acc
            return carry

        lax.fori_loop(0, BPW // L, body, 0)
        pltpu.sync_copy(out_v, out_hbm.at[pl.ds(base, BPW)])

    return k


_sc_kernel = _make_kernel()


@jax.jit
def kernel(ids, A, B):
    ids0 = ids[0].astype(jnp.int32)
    ids1 = ids[1].astype(jnp.int32)
    return _sc_kernel(ids0, ids1, A, B)
